# baseline (speedup 1.0000x reference)
import jax
import jax.numpy as jnp
from jax.experimental import pallas as pl
from jax.experimental.pallas import tpu as pltpu

_M = 512
_D = 256
_R = 16


def _blend_body(idx_ref, t_ref, v_ref, o_ref):
    i = pl.program_id(0)
    o_ref[...] = t_ref[...]
    vb = v_ref[...]
    base = (i * _R) // 128 * 128
    cols = jnp.stack([idx_ref[i * _R + r] for r in range(_R)]) - base
    sub = t_ref[:, :, pl.ds(base, 128)]
    col_ids = jax.lax.broadcasted_iota(jnp.int32, (_R, _D, 128), 2)
    mask = col_ids == cols[:, None, None]
    o_ref[:, :, pl.ds(base, 128)] = jnp.where(mask, vb[:, :, None], sub)


def kernel(t, idx, v):
    grid = _M // _R
    return pl.pallas_call(
        _blend_body,
        grid_spec=pltpu.PrefetchScalarGridSpec(
            num_scalar_prefetch=1,
            grid=(grid,),
            in_specs=[
                pl.BlockSpec((_R, _D, _M), lambda i, idx_ref: (i, 0, 0)),
                pl.BlockSpec((_R, _D), lambda i, idx_ref: (i, 0)),
            ],
            out_specs=pl.BlockSpec((_R, _D, _M), lambda i, idx_ref: (i, 0, 0)),
        ),
        out_shape=jax.ShapeDtypeStruct((_M, _D, _M), jnp.float32),
        compiler_params=pltpu.CompilerParams(
            dimension_semantics=("arbitrary",),
        ),
    )(idx, t, v)


# speedup vs baseline: 1.0007x; 1.0007x over previous
import jax
import jax.numpy as jnp
from jax.experimental import pallas as pl
from jax.experimental.pallas import tpu as pltpu

_M = 512
_D = 256
_R = 16


def _blend_body(idx_ref, t_ref, v_ref, o_ref):
    i = pl.program_id(0)
    o_ref[...] = t_ref[...]
    vb = v_ref[...]
    base = (i * _R) // 128 * 128
    cols = jnp.stack([idx_ref[i * _R + r] for r in range(_R)]) - base
    sub = t_ref[:, :, pl.ds(base, 128)]
    col_ids = jax.lax.broadcasted_iota(jnp.int32, (_R, _D, 128), 2)
    mask = col_ids == cols[:, None, None]
    o_ref[:, :, pl.ds(base, 128)] = jnp.where(mask, vb[:, :, None], sub)


def kernel(t, idx, v):
    grid = _M // _R
    return pl.pallas_call(
        _blend_body,
        grid_spec=pltpu.PrefetchScalarGridSpec(
            num_scalar_prefetch=1,
            grid=(grid,),
            in_specs=[
                pl.BlockSpec((_R, _D, _M), lambda i, idx_ref: (i, 0, 0)),
                pl.BlockSpec((_R, _D), lambda i, idx_ref: (i, 0)),
            ],
            out_specs=pl.BlockSpec((_R, _D, _M), lambda i, idx_ref: (i, 0, 0)),
        ),
        out_shape=jax.ShapeDtypeStruct((_M, _D, _M), jnp.float32),
        compiler_params=pltpu.CompilerParams(
            dimension_semantics=("parallel",),
        ),
    )(idx, t, v)
